# baseline (speedup 1.0000x reference)
import functools

import jax
import jax.numpy as jnp
from jax import lax
from jax.experimental import pallas as pl
from jax.experimental.pallas import tpu as pltpu
from jax.experimental.pallas import tpu_sc as plsc

N_CLASSES = 19
B, H, W = 8, 512, 512
N = B * H * W
K_TOP = 100000

L = 16
NSUB = 16
PER_W = N // NSUB
CHUNK = 16384
NCH = PER_W // CHUNK
UNROLL = 4
HR, HC = 128, 512

BH = 64


def _ce_body(x_ref, t_ref, o_ref):
    x = x_ref[0]
    t = t_ref[0]
    m = jnp.max(x, axis=0)
    s = jnp.sum(jnp.exp(x - m[None]), axis=0)
    cls = lax.broadcasted_iota(jnp.int32, x.shape, 0)
    xt = jnp.sum(jnp.where(cls == t[None], x, 0.0), axis=0)
    nll = jnp.log(s) + m - xt
    bits = lax.bitcast_convert_type(nll, jnp.int32)
    u = bits ^ jnp.where(bits < 0, jnp.int32(-1), jnp.int32(-2147483648))
    o_ref[...] = u.reshape(BH * W)


def _ce_keys(inputs, targets):
    return pl.pallas_call(
        _ce_body,
        grid=(B, H // BH),
        in_specs=[
            pl.BlockSpec((1, N_CLASSES, BH, W), lambda b, h: (b, 0, h, 0)),
            pl.BlockSpec((1, BH, W), lambda b, h: (b, h, 0)),
        ],
        out_specs=pl.BlockSpec((BH * W,), lambda b, h: (b * (H // BH) + h,)),
        out_shape=jax.ShapeDtypeStruct((N,), jnp.int32),
    )(inputs, targets)


def _sc_body(keys_hbm, out_hbm,
             buf0, buf1, hist, scanbuf, idxrow, totbuf, wtotbuf, decbuf,
             vec16, vec16f, outbuf, sem0, sem1,
             hist_sh, tot_sh, dec_sh, sum_sh):
    cid = lax.axis_index("c")
    sid = lax.axis_index("s")
    base = sid * PER_W

    lanes = lax.iota(jnp.int32, L)
    zeros_i = jnp.zeros((L,), jnp.int32)
    ones_i = jnp.ones((L,), jnp.int32)
    zeros_f = jnp.zeros((L,), jnp.float32)
    topbit = jnp.uint32(0x80000000)

    def val_of(u):
        bits = jnp.where(u >= topbit, u ^ topbit, ~u)
        return plsc.bitcast(bits, jnp.float32)

    for j in range(HR // L):
        idxrow[pl.ds(j * L, L)] = lanes + j * L

    def zero_scanbuf():
        @plsc.parallel_loop(0, 8 * HC // L, unroll=8)
        def _zb(i):
            scanbuf[i >> 5, pl.ds((i & 31) * L, L)] = zeros_i

    zero_scanbuf()
    pltpu.sync_copy(scanbuf, hist_sh.at[pl.ds(sid * 8, 8)])
    plsc.subcore_barrier()

    def zero_hist():
        @plsc.parallel_loop(0, HR * HC // L, unroll=8)
        def _zh(i):
            hist[i >> 5, pl.ds((i & 31) * L, L)] = zeros_i

    def stream_pass(run_chunk, carry):
        handles = {}
        handles[0] = pltpu.async_copy(
            keys_hbm.at[pl.ds(base, CHUNK)], buf0, sem0)
        for ch in range(NCH):
            cur = buf0 if ch % 2 == 0 else buf1
            if ch + 1 < NCH:
                nxt = buf1 if ch % 2 == 0 else buf0
                nsem = sem1 if ch % 2 == 0 else sem0
                handles[ch + 1] = pltpu.async_copy(
                    keys_hbm.at[pl.ds(base + (ch + 1) * CHUNK, CHUNK)],
                    nxt, nsem)
            handles[ch].wait()
            carry = run_chunk(cur, carry)
        return carry

    def p1(buf, carry):
        @plsc.parallel_loop(0, CHUNK // L, unroll=8)
        def _body(i):
            u = plsc.bitcast(buf[pl.ds(i * L, L)], jnp.uint32)
            bb = (u >> jnp.uint32(16)).astype(jnp.int32)
            plsc.addupdate_scatter(hist, [bb >> 9, bb & 511], ones_i)
        return carry

    def make_p2(d1u):
        def p2(buf, accs):
            def body(i, a):
                out = []
                for t in range(UNROLL):
                    u = plsc.bitcast(buf[pl.ds((i * UNROLL + t) * L, L)],
                                     jnp.uint32)
                    hi = u >> jnp.uint32(16)
                    bb = (u & jnp.uint32(0xFFFF)).astype(jnp.int32)
                    plsc.addupdate_scatter(
                        hist, [bb >> 9, bb & 511], ones_i, mask=(hi == d1u))
                    out.append(a[t] + jnp.where(hi > d1u, val_of(u), 0.0))
                return tuple(out)
            return plsc.parallel_loop(
                0, CHUNK // (L * UNROLL), unroll=2, carry=accs)(body)
        return p2

    def select_level(level, needed, d1u):
        weighted = level == 1
        with jax.named_scope(f"sc_merge{level}"):
            pltpu.sync_copy(hist, hist_sh.at[idxrow], add=True)
            plsc.subcore_barrier()
        pltpu.sync_copy(hist_sh.at[pl.ds(sid * 8, 8)], scanbuf)

        def bin_vals(chunk_j):
            gbin = sid * 4096 + chunk_j * L + lanes
            u_bin = (d1u << jnp.uint32(16)) | gbin.astype(jnp.uint32)
            return val_of(u_bin)

        def acc_tot(i, t):
            v = scanbuf[i >> 5, pl.ds((i & 31) * L, L)]
            if weighted:
                ti, wv = t
                return ti + jnp.sum(v), wv + jnp.sum(v.astype(jnp.float32)
                                                     * bin_vals(i))
            return t + jnp.sum(v)

        if weighted:
            tw, wsum = lax.fori_loop(0, 8 * HC // L, acc_tot,
                                     (jnp.int32(0), jnp.float32(0)))
            vec16f[...] = zeros_f + wsum
            pltpu.sync_copy(vec16f, sum_sh.at[sid])
        else:
            tw = lax.fori_loop(0, 8 * HC // L, acc_tot, jnp.int32(0))
        with jax.named_scope(f"sc_pub{level}"):
            vec16[...] = zeros_i + tw
            pltpu.sync_copy(vec16, tot_sh.at[level, sid])
            plsc.subcore_barrier()

        pltpu.sync_copy(tot_sh.at[level], totbuf)
        tv = plsc.load_gather(totbuf, [lanes, zeros_i])
        sfx_b = jnp.flip(jnp.cumsum(jnp.flip(tv)))
        mb = sfx_b >= needed
        blk = jnp.max(plsc.all_reduce_population_count(mb)) - 1
        selb = lanes == blk
        above_blocks = (jnp.sum(jnp.where(selb, sfx_b, 0))
                        - jnp.sum(jnp.where(selb, tv, 0)))
        if weighted:
            pltpu.sync_copy(sum_sh, wtotbuf)
            wv_b = plsc.load_gather(wtotbuf, [lanes, zeros_i])
            wsfx_b = jnp.flip(jnp.cumsum(jnp.flip(wv_b)))
            wabove_blocks = (jnp.sum(jnp.where(selb, wsfx_b, 0.0))
                             - jnp.sum(jnp.where(selb, wv_b, 0.0)))
        else:
            wabove_blocks = jnp.float32(0)

        @pl.when(sid == blk)
        def _owner():
            def owner_vals(chunk_j):
                gbin = blk * 4096 + chunk_j * L + lanes
                u_bin = (d1u << jnp.uint32(16)) | gbin.astype(jnp.uint32)
                return val_of(u_bin)

            def scan_chunk(i, carry):
                acc, wacc, found, binstar, aboveg, waboveg = carry
                j = 255 - i
                c = scanbuf[j >> 5, pl.ds((j & 31) * L, L)]
                sfx = jnp.flip(jnp.cumsum(jnp.flip(c)))
                m = (sfx + acc) >= needed
                cnt = jnp.max(plsc.all_reduce_population_count(m))
                this_found = cnt > 0
                newly = jnp.logical_and(this_found, found == 0)
                sel = lanes == (cnt - 1)
                sfx_at = jnp.sum(jnp.where(sel, sfx, 0))
                c_at = jnp.sum(jnp.where(sel, c, 0))
                binstar = jnp.where(newly, j * L + (cnt - 1), binstar)
                aboveg = jnp.where(newly, acc + sfx_at - c_at, aboveg)
                if weighted:
                    w = c.astype(jnp.float32) * owner_vals(j)
                    wsfx = jnp.flip(jnp.cumsum(jnp.flip(w)))
                    wsfx_at = jnp.sum(jnp.where(sel, wsfx, 0.0))
                    w_at = jnp.sum(jnp.where(sel, w, 0.0))
                    waboveg = jnp.where(newly, wacc + wsfx_at - w_at, waboveg)
                    wacc = wacc + jnp.sum(w)
                found = found | this_found.astype(jnp.int32)
                acc = acc + jnp.sum(c)
                return acc, wacc, found, binstar, aboveg, waboveg

            init = (above_blocks, wabove_blocks, jnp.int32(0), jnp.int32(0),
                    jnp.int32(0), jnp.float32(0))
            _, _, _, binstar, aboveg, waboveg = lax.fori_loop(
                0, 256, scan_chunk, init)
            digit = blk * 4096 + binstar
            nxt = needed - aboveg
            wbits = plsc.bitcast(zeros_f + waboveg, jnp.int32)
            dvec = (jnp.where(lanes == 0, digit, 0)
                    + jnp.where(lanes == 1, nxt, 0)
                    + jnp.where(lanes == 2, wbits, 0))
            vec16[...] = dvec
            pltpu.sync_copy(vec16, dec_sh.at[level])

        with jax.named_scope(f"sc_dec{level}"):
            if level == 0:
                zero_scanbuf()
                pltpu.sync_copy(scanbuf, hist_sh.at[pl.ds(sid * 8, 8)])
            plsc.subcore_barrier()
        pltpu.sync_copy(dec_sh.at[level], decbuf)
        dv = decbuf[...]
        wdv = plsc.bitcast(dv, jnp.float32)
        w_above = jnp.sum(jnp.where(lanes == 2, wdv, 0.0))
        return dv[0], dv[1], w_above

    with jax.named_scope("ph_pass1"):
        zero_hist()
        stream_pass(p1, 0)
    with jax.named_scope("ph_sel1"):
        d1, needed2, _ = select_level(0, jnp.int32(K_TOP), jnp.uint32(0))
    d1u = d1.astype(jnp.uint32)

    with jax.named_scope("ph_pass2"):
        zero_hist()
        accs = stream_pass(make_p2(d1u), (zeros_f,) * UNROLL)
        acc_out = accs[0] + accs[1] + accs[2] + accs[3]
    with jax.named_scope("ph_sel2"):
        d2, needed_f, w_above = select_level(1, needed2, d1u)

    t_u = (d1u << jnp.uint32(16)) | d2.astype(jnp.uint32)

    local = jnp.sum(acc_out)
    vec16f[...] = zeros_f + local
    pltpu.sync_copy(vec16f, sum_sh.at[sid])
    plsc.subcore_barrier()

    @pl.when(jnp.logical_and(sid == 0, cid == 0))
    def _emit():
        pltpu.sync_copy(sum_sh, wtotbuf)
        sums = plsc.load_gather(wtotbuf, [lanes, zeros_i])
        s_above = jnp.sum(sums) + w_above
        t_val = val_of(jnp.zeros((L,), jnp.uint32) + t_u)
        need_f = needed_f.astype(jnp.float32)
        res = (zeros_f + s_above + need_f * t_val) / jnp.float32(K_TOP)
        outbuf[...] = res
        pltpu.sync_copy(outbuf, out_hbm)


def _topk_mean(keys_flat):
    mesh = plsc.VectorSubcoreMesh(
        core_axis_name="c", subcore_axis_name="s", num_cores=2,
        num_subcores=NSUB)
    fn = functools.partial(
        pl.kernel,
        out_type=jax.ShapeDtypeStruct((L,), jnp.float32),
        mesh=mesh,
        compiler_params=pltpu.CompilerParams(
            use_tc_tiling_on_sc=False, needs_layout_passes=False),
        scratch_types=[
            pltpu.VMEM((CHUNK,), jnp.int32),
            pltpu.VMEM((CHUNK,), jnp.int32),
            pltpu.VMEM((HR, HC), jnp.int32),
            pltpu.VMEM((8, HC), jnp.int32),
            pltpu.VMEM((HR,), jnp.int32),
            pltpu.VMEM((L, L), jnp.int32),
            pltpu.VMEM((L, L), jnp.float32),
            pltpu.VMEM((L,), jnp.int32),
            pltpu.VMEM((L,), jnp.int32),
            pltpu.VMEM((L,), jnp.float32),
            pltpu.VMEM((L,), jnp.float32),
            pltpu.SemaphoreType.DMA,
            pltpu.SemaphoreType.DMA,
            pltpu.VMEM_SHARED((HR, HC), jnp.int32),
            pltpu.VMEM_SHARED((2, L, L), jnp.int32),
            pltpu.VMEM_SHARED((2, L), jnp.int32),
            pltpu.VMEM_SHARED((L, L), jnp.float32),
        ],
    )(_sc_body)
    return fn(keys_flat)


def kernel(inputs, targets):
    keys = _ce_keys(inputs, targets)
    out = _topk_mean(keys)
    return out[0]


# speedup vs baseline: 1.0224x; 1.0224x over previous
import functools

import jax
import jax.numpy as jnp
from jax import lax
from jax.experimental import pallas as pl
from jax.experimental.pallas import tpu as pltpu
from jax.experimental.pallas import tpu_sc as plsc

N_CLASSES = 19
B, H, W = 8, 512, 512
N = B * H * W
K_TOP = 100000

L = 16
NSUB = 16
PER_W = N // NSUB
CHUNK = 16384
NCH = PER_W // CHUNK
UNROLL = 4
HR, HC = 128, 512

BH = 64


def _ce_body(x_ref, t_ref, o_ref):
    x = x_ref[0]
    t = t_ref[0]
    m = jnp.max(x, axis=0)
    s = jnp.sum(jnp.exp(x - m[None]), axis=0)
    cls = lax.broadcasted_iota(jnp.int32, x.shape, 0)
    xt = jnp.sum(jnp.where(cls == t[None], x, 0.0), axis=0)
    nll = jnp.log(s) + m - xt
    bits = lax.bitcast_convert_type(nll, jnp.int32)
    u = bits ^ jnp.where(bits < 0, jnp.int32(-1), jnp.int32(-2147483648))
    o_ref[...] = u.reshape(BH * W)


def _ce_keys(inputs, targets):
    return pl.pallas_call(
        _ce_body,
        grid=(B, H // BH),
        in_specs=[
            pl.BlockSpec((1, N_CLASSES, BH, W), lambda b, h: (b, 0, h, 0)),
            pl.BlockSpec((1, BH, W), lambda b, h: (b, h, 0)),
        ],
        out_specs=pl.BlockSpec((BH * W,), lambda b, h: (b * (H // BH) + h,)),
        out_shape=jax.ShapeDtypeStruct((N,), jnp.int32),
    )(inputs, targets)


def _sc_body(keys_hbm, out_hbm,
             buf0, buf1, hist, scanbuf, idxrow, totbuf, wtotbuf, decbuf,
             vec16, vec16f, outbuf, sem0, sem1,
             hist_sh0, hist_sh1, tot_sh, dec_sh, sum_sh):
    cid = lax.axis_index("c")
    sid = lax.axis_index("s")
    base = sid * PER_W

    lanes = lax.iota(jnp.int32, L)
    zeros_i = jnp.zeros((L,), jnp.int32)
    ones_i = jnp.ones((L,), jnp.int32)
    zeros_f = jnp.zeros((L,), jnp.float32)
    topbit = jnp.uint32(0x80000000)

    def val_of(u):
        bits = jnp.where(u >= topbit, u ^ topbit, ~u)
        return plsc.bitcast(bits, jnp.float32)

    first_h = pltpu.async_copy(keys_hbm.at[pl.ds(base, CHUNK)], buf0, sem0)

    for j in range(HR // L):
        idxrow[pl.ds(j * L, L)] = lanes + j * L

    def zero_scanbuf():
        @plsc.parallel_loop(0, 8 * HC // L, unroll=8)
        def _zb(i):
            scanbuf[i >> 5, pl.ds((i & 31) * L, L)] = zeros_i

    zero_scanbuf()
    pltpu.sync_copy(scanbuf, hist_sh0.at[pl.ds(sid * 8, 8)])
    pltpu.sync_copy(scanbuf, hist_sh1.at[pl.ds(sid * 8, 8)])
    plsc.subcore_barrier()

    def zero_hist():
        @plsc.parallel_loop(0, HR * HC // L, unroll=8)
        def _zh(i):
            hist[i >> 5, pl.ds((i & 31) * L, L)] = zeros_i

    def stream_pass(run_chunk, carry, h0):
        handles = {0: h0}
        for ch in range(NCH):
            cur = buf0 if ch % 2 == 0 else buf1
            if ch + 1 < NCH:
                nxt = buf1 if ch % 2 == 0 else buf0
                nsem = sem1 if ch % 2 == 0 else sem0
                handles[ch + 1] = pltpu.async_copy(
                    keys_hbm.at[pl.ds(base + (ch + 1) * CHUNK, CHUNK)],
                    nxt, nsem)
            handles[ch].wait()
            carry = run_chunk(cur, carry)
        return carry

    def p1(buf, carry):
        @plsc.parallel_loop(0, CHUNK // L, unroll=8)
        def _body(i):
            u = plsc.bitcast(buf[pl.ds(i * L, L)], jnp.uint32)
            bb = (u >> jnp.uint32(16)).astype(jnp.int32)
            plsc.addupdate_scatter(hist, [bb >> 9, bb & 511], ones_i)
        return carry

    def make_p2(d1u):
        def p2(buf, accs):
            def body(i, a):
                out = []
                for t in range(UNROLL):
                    u = plsc.bitcast(buf[pl.ds((i * UNROLL + t) * L, L)],
                                     jnp.uint32)
                    hi = u >> jnp.uint32(16)
                    bb = (u & jnp.uint32(0xFFFF)).astype(jnp.int32)
                    plsc.addupdate_scatter(
                        hist, [bb >> 9, bb & 511], ones_i, mask=(hi == d1u))
                    out.append(a[t] + jnp.where(hi > d1u, val_of(u), 0.0))
                return tuple(out)
            return plsc.parallel_loop(
                0, CHUNK // (L * UNROLL), unroll=2, carry=accs)(body)
        return p2

    def select_level(level, needed, d1u, hist_sh):
        weighted = level == 1
        with jax.named_scope(f"sc_merge{level}"):
            pltpu.sync_copy(hist, hist_sh.at[idxrow], add=True)
            plsc.subcore_barrier()
        pltpu.sync_copy(hist_sh.at[pl.ds(sid * 8, 8)], scanbuf)

        def bin_vals(chunk_j):
            gbin = sid * 4096 + chunk_j * L + lanes
            u_bin = (d1u << jnp.uint32(16)) | gbin.astype(jnp.uint32)
            return val_of(u_bin)

        def acc_tot(i, t):
            v = scanbuf[i >> 5, pl.ds((i & 31) * L, L)]
            if weighted:
                ti, wv = t
                return ti + jnp.sum(v), wv + jnp.sum(v.astype(jnp.float32)
                                                     * bin_vals(i))
            return t + jnp.sum(v)

        if weighted:
            tw, wsum = lax.fori_loop(0, 8 * HC // L, acc_tot,
                                     (jnp.int32(0), jnp.float32(0)))
            vec16f[...] = zeros_f + wsum
            pltpu.sync_copy(vec16f, sum_sh.at[sid])
        else:
            tw = lax.fori_loop(0, 8 * HC // L, acc_tot, jnp.int32(0))
        with jax.named_scope(f"sc_pub{level}"):
            vec16[...] = zeros_i + tw
            pltpu.sync_copy(vec16, tot_sh.at[level, sid])
            plsc.subcore_barrier()

        pltpu.sync_copy(tot_sh.at[level], totbuf)
        tv = plsc.load_gather(totbuf, [lanes, zeros_i])
        sfx_b = jnp.flip(jnp.cumsum(jnp.flip(tv)))
        mb = sfx_b >= needed
        blk = jnp.max(plsc.all_reduce_population_count(mb)) - 1
        selb = lanes == blk
        above_blocks = (jnp.sum(jnp.where(selb, sfx_b, 0))
                        - jnp.sum(jnp.where(selb, tv, 0)))
        if weighted:
            pltpu.sync_copy(sum_sh, wtotbuf)
            wv_b = plsc.load_gather(wtotbuf, [lanes, zeros_i])
            wsfx_b = jnp.flip(jnp.cumsum(jnp.flip(wv_b)))
            wabove_blocks = (jnp.sum(jnp.where(selb, wsfx_b, 0.0))
                             - jnp.sum(jnp.where(selb, wv_b, 0.0)))
        else:
            wabove_blocks = jnp.float32(0)

        @pl.when(sid == blk)
        def _owner():
            def owner_vals(chunk_j):
                gbin = blk * 4096 + chunk_j * L + lanes
                u_bin = (d1u << jnp.uint32(16)) | gbin.astype(jnp.uint32)
                return val_of(u_bin)

            def scan_chunk(i, carry):
                acc, wacc, found, binstar, aboveg, waboveg = carry
                j = 255 - i
                c = scanbuf[j >> 5, pl.ds((j & 31) * L, L)]
                sfx = jnp.flip(jnp.cumsum(jnp.flip(c)))
                m = (sfx + acc) >= needed
                cnt = jnp.max(plsc.all_reduce_population_count(m))
                this_found = cnt > 0
                newly = jnp.logical_and(this_found, found == 0)
                sel = lanes == (cnt - 1)
                sfx_at = jnp.sum(jnp.where(sel, sfx, 0))
                c_at = jnp.sum(jnp.where(sel, c, 0))
                binstar = jnp.where(newly, j * L + (cnt - 1), binstar)
                aboveg = jnp.where(newly, acc + sfx_at - c_at, aboveg)
                if weighted:
                    w = c.astype(jnp.float32) * owner_vals(j)
                    wsfx = jnp.flip(jnp.cumsum(jnp.flip(w)))
                    wsfx_at = jnp.sum(jnp.where(sel, wsfx, 0.0))
                    w_at = jnp.sum(jnp.where(sel, w, 0.0))
                    waboveg = jnp.where(newly, wacc + wsfx_at - w_at, waboveg)
                    wacc = wacc + jnp.sum(w)
                found = found | this_found.astype(jnp.int32)
                acc = acc + jnp.sum(c)
                return acc, wacc, found, binstar, aboveg, waboveg

            init = (above_blocks, wabove_blocks, jnp.int32(0), jnp.int32(0),
                    jnp.int32(0), jnp.float32(0))
            _, _, _, binstar, aboveg, waboveg = lax.fori_loop(
                0, 256, scan_chunk, init)
            digit = blk * 4096 + binstar
            nxt = needed - aboveg
            wbits = plsc.bitcast(zeros_f + waboveg, jnp.int32)
            dvec = (jnp.where(lanes == 0, digit, 0)
                    + jnp.where(lanes == 1, nxt, 0)
                    + jnp.where(lanes == 2, wbits, 0))
            vec16[...] = dvec
            pltpu.sync_copy(vec16, dec_sh.at[level])

        with jax.named_scope(f"sc_dec{level}"):
            plsc.subcore_barrier()
        pltpu.sync_copy(dec_sh.at[level], decbuf)
        dv = decbuf[...]
        wdv = plsc.bitcast(dv, jnp.float32)
        w_above = jnp.sum(jnp.where(lanes == 2, wdv, 0.0))
        return dv[0], dv[1], w_above

    with jax.named_scope("ph_pass1"):
        zero_hist()
        stream_pass(p1, 0, first_h)
        h2 = pltpu.async_copy(keys_hbm.at[pl.ds(base, CHUNK)], buf0, sem0)
    with jax.named_scope("ph_sel1"):
        d1, needed2, _ = select_level(0, jnp.int32(K_TOP), jnp.uint32(0),
                                      hist_sh0)
    d1u = d1.astype(jnp.uint32)

    with jax.named_scope("ph_pass2"):
        zero_hist()
        accs = stream_pass(make_p2(d1u), (zeros_f,) * UNROLL, h2)
        acc_out = accs[0] + accs[1] + accs[2] + accs[3]
    with jax.named_scope("ph_sel2"):
        d2, needed_f, w_above = select_level(1, needed2, d1u, hist_sh1)

    t_u = (d1u << jnp.uint32(16)) | d2.astype(jnp.uint32)

    local = jnp.sum(acc_out)
    vec16f[...] = zeros_f + local
    pltpu.sync_copy(vec16f, sum_sh.at[sid])
    plsc.subcore_barrier()

    @pl.when(jnp.logical_and(sid == 0, cid == 0))
    def _emit():
        pltpu.sync_copy(sum_sh, wtotbuf)
        sums = plsc.load_gather(wtotbuf, [lanes, zeros_i])
        s_above = jnp.sum(sums) + w_above
        t_val = val_of(jnp.zeros((L,), jnp.uint32) + t_u)
        need_f = needed_f.astype(jnp.float32)
        res = (zeros_f + s_above + need_f * t_val) / jnp.float32(K_TOP)
        outbuf[...] = res
        pltpu.sync_copy(outbuf, out_hbm)


def _topk_mean(keys_flat):
    mesh = plsc.VectorSubcoreMesh(
        core_axis_name="c", subcore_axis_name="s", num_cores=2,
        num_subcores=NSUB)
    fn = functools.partial(
        pl.kernel,
        out_type=jax.ShapeDtypeStruct((L,), jnp.float32),
        mesh=mesh,
        compiler_params=pltpu.CompilerParams(
            use_tc_tiling_on_sc=False, needs_layout_passes=False),
        scratch_types=[
            pltpu.VMEM((CHUNK,), jnp.int32),
            pltpu.VMEM((CHUNK,), jnp.int32),
            pltpu.VMEM((HR, HC), jnp.int32),
            pltpu.VMEM((8, HC), jnp.int32),
            pltpu.VMEM((HR,), jnp.int32),
            pltpu.VMEM((L, L), jnp.int32),
            pltpu.VMEM((L, L), jnp.float32),
            pltpu.VMEM((L,), jnp.int32),
            pltpu.VMEM((L,), jnp.int32),
            pltpu.VMEM((L,), jnp.float32),
            pltpu.VMEM((L,), jnp.float32),
            pltpu.SemaphoreType.DMA,
            pltpu.SemaphoreType.DMA,
            pltpu.VMEM_SHARED((HR, HC), jnp.int32),
            pltpu.VMEM_SHARED((HR, HC), jnp.int32),
            pltpu.VMEM_SHARED((2, L, L), jnp.int32),
            pltpu.VMEM_SHARED((2, L), jnp.int32),
            pltpu.VMEM_SHARED((L, L), jnp.float32),
        ],
    )(_sc_body)
    return fn(keys_flat)


def kernel(inputs, targets):
    keys = _ce_keys(inputs, targets)
    out = _topk_mean(keys)
    return out[0]


# speedup vs baseline: 1.0233x; 1.0008x over previous
import functools

import jax
import jax.numpy as jnp
from jax import lax
from jax.experimental import pallas as pl
from jax.experimental.pallas import tpu as pltpu
from jax.experimental.pallas import tpu_sc as plsc

N_CLASSES = 19
B, H, W = 8, 512, 512
N = B * H * W
K_TOP = 100000

L = 16
NSUB = 16
PER_W = N // NSUB
CHUNK = 16384
NCH = PER_W // CHUNK
UNROLL = 4
HR, HC = 128, 512

BH = 64


def _ce_body(x_ref, t_ref, o_ref):
    x = x_ref[0]
    t = t_ref[0]
    m = jnp.max(x, axis=0)
    s = jnp.sum(jnp.exp(x - m[None]), axis=0)
    cls = lax.broadcasted_iota(jnp.int32, x.shape, 0)
    xt = jnp.sum(jnp.where(cls == t[None], x, 0.0), axis=0)
    nll = jnp.log(s) + m - xt
    bits = lax.bitcast_convert_type(nll, jnp.int32)
    u = bits ^ jnp.where(bits < 0, jnp.int32(-1), jnp.int32(-2147483648))
    o_ref[...] = u.reshape(BH * W)


def _ce_keys(inputs, targets):
    return pl.pallas_call(
        _ce_body,
        grid=(B, H // BH),
        in_specs=[
            pl.BlockSpec((1, N_CLASSES, BH, W), lambda b, h: (b, 0, h, 0)),
            pl.BlockSpec((1, BH, W), lambda b, h: (b, h, 0)),
        ],
        out_specs=pl.BlockSpec((BH * W,), lambda b, h: (b * (H // BH) + h,)),
        out_shape=jax.ShapeDtypeStruct((N,), jnp.int32),
    )(inputs, targets)


def _sc_body(keys_hbm, out_hbm,
             buf0, buf1, hist, scanbuf, idxrow, totbuf, wtotbuf, decbuf,
             vec16, vec16f, outbuf, sem0, sem1,
             hist_sh0, hist_sh1, tot_sh, dec_sh, sum_sh):
    cid = lax.axis_index("c")
    sid = lax.axis_index("s")
    base = sid * PER_W

    lanes = lax.iota(jnp.int32, L)
    zeros_i = jnp.zeros((L,), jnp.int32)
    ones_i = jnp.ones((L,), jnp.int32)
    zeros_f = jnp.zeros((L,), jnp.float32)
    topbit = jnp.uint32(0x80000000)

    def val_of(u):
        bits = jnp.where(u >= topbit, u ^ topbit, ~u)
        return plsc.bitcast(bits, jnp.float32)

    first_h = pltpu.async_copy(keys_hbm.at[pl.ds(base, CHUNK)], buf0, sem0)

    for j in range(HR // L):
        idxrow[pl.ds(j * L, L)] = lanes + j * L

    def zero_scanbuf():
        @plsc.parallel_loop(0, 8 * HC // L, unroll=8)
        def _zb(i):
            scanbuf[i >> 5, pl.ds((i & 31) * L, L)] = zeros_i

    zero_scanbuf()
    pltpu.sync_copy(scanbuf, hist_sh0.at[pl.ds(sid * 8, 8)])
    pltpu.sync_copy(scanbuf, hist_sh1.at[pl.ds(sid * 8, 8)])
    plsc.subcore_barrier()

    def zero_hist():
        @plsc.parallel_loop(0, HR * HC // L, unroll=8)
        def _zh(i):
            hist[i >> 5, pl.ds((i & 31) * L, L)] = zeros_i

    def stream_pass(run_chunk, carry, h0):
        handles = {0: h0}
        for ch in range(NCH):
            cur = buf0 if ch % 2 == 0 else buf1
            if ch + 1 < NCH:
                nxt = buf1 if ch % 2 == 0 else buf0
                nsem = sem1 if ch % 2 == 0 else sem0
                handles[ch + 1] = pltpu.async_copy(
                    keys_hbm.at[pl.ds(base + (ch + 1) * CHUNK, CHUNK)],
                    nxt, nsem)
            handles[ch].wait()
            carry = run_chunk(cur, carry)
        return carry

    def p1(buf, carry):
        @plsc.parallel_loop(0, CHUNK // L, unroll=8)
        def _body(i):
            u = plsc.bitcast(buf[pl.ds(i * L, L)], jnp.uint32)
            bb = (u >> jnp.uint32(16)).astype(jnp.int32)
            plsc.addupdate_scatter(hist, [bb >> 9, bb & 511], ones_i)
        return carry

    def make_p2(d1u):
        def p2(buf, accs):
            def body(i, a):
                out = []
                for t in range(UNROLL):
                    u = plsc.bitcast(buf[pl.ds((i * UNROLL + t) * L, L)],
                                     jnp.uint32)
                    hi = u >> jnp.uint32(16)
                    bb = (u & jnp.uint32(0xFFFF)).astype(jnp.int32)
                    plsc.addupdate_scatter(
                        hist, [bb >> 9, bb & 511], ones_i, mask=(hi == d1u))
                    out.append(a[t] + jnp.where(hi > d1u, val_of(u), 0.0))
                return tuple(out)
            return plsc.parallel_loop(
                0, CHUNK // (L * UNROLL), unroll=2, carry=accs)(body)
        return p2

    def select_level(level, needed, d1u, hist_sh):
        weighted = level == 1
        with jax.named_scope(f"sc_merge{level}"):
            pltpu.sync_copy(hist, hist_sh.at[idxrow], add=True)
            plsc.subcore_barrier()
        pltpu.sync_copy(hist_sh.at[pl.ds(sid * 8, 8)], scanbuf)

        def bin_vals(chunk_j):
            gbin = sid * 4096 + chunk_j * L + lanes
            u_bin = (d1u << jnp.uint32(16)) | gbin.astype(jnp.uint32)
            return val_of(u_bin)

        def acc_tot(i, t):
            v = scanbuf[i >> 5, pl.ds((i & 31) * L, L)]
            if weighted:
                ti, wv = t
                return ti + jnp.sum(v), wv + jnp.sum(v.astype(jnp.float32)
                                                     * bin_vals(i))
            return t + jnp.sum(v)

        if weighted:
            tw, wsum = lax.fori_loop(0, 8 * HC // L, acc_tot,
                                     (jnp.int32(0), jnp.float32(0)))
            vec16f[...] = zeros_f + wsum
            pltpu.sync_copy(vec16f, sum_sh.at[sid])
        else:
            tw = lax.fori_loop(0, 8 * HC // L, acc_tot, jnp.int32(0))
        with jax.named_scope(f"sc_pub{level}"):
            vec16[...] = zeros_i + tw
            pltpu.sync_copy(vec16, tot_sh.at[level, sid])
            plsc.subcore_barrier()

        pltpu.sync_copy(tot_sh.at[level], totbuf)
        tv = plsc.load_gather(totbuf, [lanes, zeros_i])
        sfx_b = jnp.flip(jnp.cumsum(jnp.flip(tv)))
        mb = sfx_b >= needed
        blk = jnp.max(plsc.all_reduce_population_count(mb)) - 1
        selb = lanes == blk
        above_blocks = (jnp.sum(jnp.where(selb, sfx_b, 0))
                        - jnp.sum(jnp.where(selb, tv, 0)))
        if weighted:
            pltpu.sync_copy(sum_sh, wtotbuf)
            wv_b = plsc.load_gather(wtotbuf, [lanes, zeros_i])
            wsfx_b = jnp.flip(jnp.cumsum(jnp.flip(wv_b)))
            wabove_blocks = (jnp.sum(jnp.where(selb, wsfx_b, 0.0))
                             - jnp.sum(jnp.where(selb, wv_b, 0.0)))
        else:
            wabove_blocks = jnp.float32(0)

        @pl.when(sid == blk)
        @jax.named_scope(f"sc_scan{level}")
        def _owner():
            def owner_vals(chunk_j):
                gbin = blk * 4096 + chunk_j * L + lanes
                u_bin = (d1u << jnp.uint32(16)) | gbin.astype(jnp.uint32)
                return val_of(u_bin)

            def scan_chunk(i, carry):
                acc, wacc, found, binstar, aboveg, waboveg = carry
                j = 255 - i
                c = scanbuf[j >> 5, pl.ds((j & 31) * L, L)]
                sfx = jnp.flip(jnp.cumsum(jnp.flip(c)))
                m = (sfx + acc) >= needed
                cnt = jnp.max(plsc.all_reduce_population_count(m))
                this_found = cnt > 0
                newly = jnp.logical_and(this_found, found == 0)
                sel = lanes == (cnt - 1)
                sfx_at = jnp.sum(jnp.where(sel, sfx, 0))
                c_at = jnp.sum(jnp.where(sel, c, 0))
                binstar = jnp.where(newly, j * L + (cnt - 1), binstar)
                aboveg = jnp.where(newly, acc + sfx_at - c_at, aboveg)
                if weighted:
                    w = c.astype(jnp.float32) * owner_vals(j)
                    wsfx = jnp.flip(jnp.cumsum(jnp.flip(w)))
                    wsfx_at = jnp.sum(jnp.where(sel, wsfx, 0.0))
                    w_at = jnp.sum(jnp.where(sel, w, 0.0))
                    waboveg = jnp.where(newly, wacc + wsfx_at - w_at, waboveg)
                    wacc = wacc + jnp.sum(w)
                found = found | this_found.astype(jnp.int32)
                acc = acc + jnp.sum(c)
                return acc, wacc, found, binstar, aboveg, waboveg

            init = (above_blocks, wabove_blocks, jnp.int32(0), jnp.int32(0),
                    jnp.int32(0), jnp.float32(0))
            _, _, _, binstar, aboveg, waboveg = lax.fori_loop(
                0, 256, scan_chunk, init)
            digit = blk * 4096 + binstar
            nxt = needed - aboveg
            wbits = plsc.bitcast(zeros_f + waboveg, jnp.int32)
            dvec = (jnp.where(lanes == 0, digit, 0)
                    + jnp.where(lanes == 1, nxt, 0)
                    + jnp.where(lanes == 2, wbits, 0))
            vec16[...] = dvec
            pltpu.sync_copy(vec16, dec_sh.at[level])

        with jax.named_scope(f"sc_dec{level}"):
            plsc.subcore_barrier()
        pltpu.sync_copy(dec_sh.at[level], decbuf)
        dv = decbuf[...]
        wdv = plsc.bitcast(dv, jnp.float32)
        w_above = jnp.sum(jnp.where(lanes == 2, wdv, 0.0))
        return dv[0], dv[1], w_above

    with jax.named_scope("ph_pass1"):
        zero_hist()
        stream_pass(p1, 0, first_h)
        h2 = pltpu.async_copy(keys_hbm.at[pl.ds(base, CHUNK)], buf0, sem0)
    with jax.named_scope("ph_sel1"):
        d1, needed2, _ = select_level(0, jnp.int32(K_TOP), jnp.uint32(0),
                                      hist_sh0)
    d1u = d1.astype(jnp.uint32)

    with jax.named_scope("ph_pass2"):
        zero_hist()
        accs = stream_pass(make_p2(d1u), (zeros_f,) * UNROLL, h2)
        acc_out = accs[0] + accs[1] + accs[2] + accs[3]
    with jax.named_scope("ph_sel2"):
        d2, needed_f, w_above = select_level(1, needed2, d1u, hist_sh1)

    t_u = (d1u << jnp.uint32(16)) | d2.astype(jnp.uint32)

    local = jnp.sum(acc_out)
    vec16f[...] = zeros_f + local
    pltpu.sync_copy(vec16f, sum_sh.at[sid])
    plsc.subcore_barrier()

    @pl.when(jnp.logical_and(sid == 0, cid == 0))
    def _emit():
        pltpu.sync_copy(sum_sh, wtotbuf)
        sums = plsc.load_gather(wtotbuf, [lanes, zeros_i])
        s_above = jnp.sum(sums) + w_above
        t_val = val_of(jnp.zeros((L,), jnp.uint32) + t_u)
        need_f = needed_f.astype(jnp.float32)
        res = (zeros_f + s_above + need_f * t_val) / jnp.float32(K_TOP)
        outbuf[...] = res
        pltpu.sync_copy(outbuf, out_hbm)


def _topk_mean(keys_flat):
    mesh = plsc.VectorSubcoreMesh(
        core_axis_name="c", subcore_axis_name="s", num_cores=2,
        num_subcores=NSUB)
    fn = functools.partial(
        pl.kernel,
        out_type=jax.ShapeDtypeStruct((L,), jnp.float32),
        mesh=mesh,
        compiler_params=pltpu.CompilerParams(
            use_tc_tiling_on_sc=False, needs_layout_passes=False),
        scratch_types=[
            pltpu.VMEM((CHUNK,), jnp.int32),
            pltpu.VMEM((CHUNK,), jnp.int32),
            pltpu.VMEM((HR, HC), jnp.int32),
            pltpu.VMEM((8, HC), jnp.int32),
            pltpu.VMEM((HR,), jnp.int32),
            pltpu.VMEM((L, L), jnp.int32),
            pltpu.VMEM((L, L), jnp.float32),
            pltpu.VMEM((L,), jnp.int32),
            pltpu.VMEM((L,), jnp.int32),
            pltpu.VMEM((L,), jnp.float32),
            pltpu.VMEM((L,), jnp.float32),
            pltpu.SemaphoreType.DMA,
            pltpu.SemaphoreType.DMA,
            pltpu.VMEM_SHARED((HR, HC), jnp.int32),
            pltpu.VMEM_SHARED((HR, HC), jnp.int32),
            pltpu.VMEM_SHARED((2, L, L), jnp.int32),
            pltpu.VMEM_SHARED((2, L), jnp.int32),
            pltpu.VMEM_SHARED((L, L), jnp.float32),
        ],
    )(_sc_body)
    return fn(keys_flat)


def kernel(inputs, targets):
    keys = _ce_keys(inputs, targets)
    out = _topk_mean(keys)
    return out[0]


# speedup vs baseline: 1.0884x; 1.0636x over previous
import functools

import jax
import jax.numpy as jnp
from jax import lax
from jax.experimental import pallas as pl
from jax.experimental.pallas import tpu as pltpu
from jax.experimental.pallas import tpu_sc as plsc

N_CLASSES = 19
B, H, W = 8, 512, 512
N = B * H * W
K_TOP = 100000

L = 16
NSUB = 16
PER_W = N // NSUB
CHUNK = 16384
NCH = PER_W // CHUNK
UNROLL = 4
HR, HC = 128, 512

BH = 64


def _ce_body(x_ref, t_ref, o_ref):
    x = x_ref[0]
    t = t_ref[0]
    m = jnp.max(x, axis=0)
    s = jnp.sum(jnp.exp(x - m[None]), axis=0)
    cls = lax.broadcasted_iota(jnp.int32, x.shape, 0)
    xt = jnp.sum(jnp.where(cls == t[None], x, 0.0), axis=0)
    nll = jnp.log(s) + m - xt
    bits = lax.bitcast_convert_type(nll, jnp.int32)
    u = bits ^ jnp.where(bits < 0, jnp.int32(-1), jnp.int32(-2147483648))
    o_ref[...] = u.reshape(BH * W)


def _ce_keys(inputs, targets):
    return pl.pallas_call(
        _ce_body,
        grid=(B, H // BH),
        in_specs=[
            pl.BlockSpec((1, N_CLASSES, BH, W), lambda b, h: (b, 0, h, 0)),
            pl.BlockSpec((1, BH, W), lambda b, h: (b, h, 0)),
        ],
        out_specs=pl.BlockSpec((BH * W,), lambda b, h: (b * (H // BH) + h,)),
        out_shape=jax.ShapeDtypeStruct((N,), jnp.int32),
    )(inputs, targets)


def _sc_body(keys_hbm, out_hbm,
             buf0, buf1, hist, scanbuf, idxrow, totbuf, wtotbuf, decbuf,
             vec16, vec16f, outbuf, ctot, wctot, sem0, sem1,
             hist_sh0, hist_sh1, tot_sh, dec_sh, sum_sh):
    cid = lax.axis_index("c")
    sid = lax.axis_index("s")
    base = sid * PER_W

    lanes = lax.iota(jnp.int32, L)
    zeros_i = jnp.zeros((L,), jnp.int32)
    ones_i = jnp.ones((L,), jnp.int32)
    zeros_f = jnp.zeros((L,), jnp.float32)
    topbit = jnp.uint32(0x80000000)

    def val_of(u):
        bits = jnp.where(u >= topbit, u ^ topbit, ~u)
        return plsc.bitcast(bits, jnp.float32)

    first_h = pltpu.async_copy(keys_hbm.at[pl.ds(base, CHUNK)], buf0, sem0)

    for j in range(HR // L):
        idxrow[pl.ds(j * L, L)] = lanes + j * L

    def zero_scanbuf():
        @plsc.parallel_loop(0, 8 * HC // L, unroll=8)
        def _zb(i):
            scanbuf[i >> 5, pl.ds((i & 31) * L, L)] = zeros_i

    zero_scanbuf()
    pltpu.sync_copy(scanbuf, hist_sh0.at[pl.ds(sid * 8, 8)])
    pltpu.sync_copy(scanbuf, hist_sh1.at[pl.ds(sid * 8, 8)])
    plsc.subcore_barrier()

    def zero_hist():
        @plsc.parallel_loop(0, HR * HC // L, unroll=8)
        def _zh(i):
            hist[i >> 5, pl.ds((i & 31) * L, L)] = zeros_i

    def stream_pass(run_chunk, carry, h0):
        handles = {0: h0}
        for ch in range(NCH):
            cur = buf0 if ch % 2 == 0 else buf1
            if ch + 1 < NCH:
                nxt = buf1 if ch % 2 == 0 else buf0
                nsem = sem1 if ch % 2 == 0 else sem0
                handles[ch + 1] = pltpu.async_copy(
                    keys_hbm.at[pl.ds(base + (ch + 1) * CHUNK, CHUNK)],
                    nxt, nsem)
            handles[ch].wait()
            carry = run_chunk(cur, carry)
        return carry

    def p1(buf, carry):
        @plsc.parallel_loop(0, CHUNK // L, unroll=8)
        def _body(i):
            u = plsc.bitcast(buf[pl.ds(i * L, L)], jnp.uint32)
            bb = (u >> jnp.uint32(16)).astype(jnp.int32)
            plsc.addupdate_scatter(hist, [bb >> 9, bb & 511], ones_i)
        return carry

    def make_p2(d1u):
        def p2(buf, accs):
            def body(i, a):
                out = []
                for t in range(UNROLL):
                    u = plsc.bitcast(buf[pl.ds((i * UNROLL + t) * L, L)],
                                     jnp.uint32)
                    hi = u >> jnp.uint32(16)
                    bb = (u & jnp.uint32(0xFFFF)).astype(jnp.int32)
                    plsc.addupdate_scatter(
                        hist, [bb >> 9, bb & 511], ones_i, mask=(hi == d1u))
                    out.append(a[t] + jnp.where(hi > d1u, val_of(u), 0.0))
                return tuple(out)
            return plsc.parallel_loop(
                0, CHUNK // (L * UNROLL), unroll=2, carry=accs)(body)
        return p2

    def select_level(level, needed, d1u, hist_sh):
        weighted = level == 1
        with jax.named_scope(f"sc_merge{level}"):
            pltpu.sync_copy(hist, hist_sh.at[idxrow], add=True)
            plsc.subcore_barrier()
        pltpu.sync_copy(hist_sh.at[pl.ds(sid * 8, 8)], scanbuf)

        def bin_vals(chunk_j):
            gbin = sid * 4096 + chunk_j * L + lanes
            u_bin = (d1u << jnp.uint32(16)) | gbin.astype(jnp.uint32)
            return val_of(u_bin)

        def acc_tot(i, carry):
            v = scanbuf[i >> 5, pl.ds((i & 31) * L, L)]
            s = jnp.sum(v)
            lane_sel = lanes == (i & 15)
            if weighted:
                t, wv, tvec, wvec = carry
                w = v.astype(jnp.float32) * bin_vals(i)
                ws = jnp.sum(w)
                wvec = jnp.where(lane_sel, ws, wvec)
                wctot[pl.ds((i >> 4) * L, L)] = wvec
                tvec = jnp.where(lane_sel, s, tvec)
                ctot[pl.ds((i >> 4) * L, L)] = tvec
                return t + s, wv + ws, tvec, wvec
            t, tvec = carry
            tvec = jnp.where(lane_sel, s, tvec)
            ctot[pl.ds((i >> 4) * L, L)] = tvec
            return t + s, tvec

        if weighted:
            tw, wsum, _, _ = lax.fori_loop(
                0, 8 * HC // L, acc_tot,
                (jnp.int32(0), jnp.float32(0), zeros_i, zeros_f))
            vec16f[...] = zeros_f + wsum
            pltpu.sync_copy(vec16f, sum_sh.at[sid])
        else:
            tw, _ = lax.fori_loop(0, 8 * HC // L, acc_tot,
                                  (jnp.int32(0), zeros_i))
        with jax.named_scope(f"sc_pub{level}"):
            vec16[...] = zeros_i + tw
            pltpu.sync_copy(vec16, tot_sh.at[level, sid])
            plsc.subcore_barrier()

        pltpu.sync_copy(tot_sh.at[level], totbuf)
        tv = plsc.load_gather(totbuf, [lanes, zeros_i])
        sfx_b = jnp.flip(jnp.cumsum(jnp.flip(tv)))
        mb = sfx_b >= needed
        blk = jnp.max(plsc.all_reduce_population_count(mb)) - 1
        selb = lanes == blk
        above_blocks = (jnp.sum(jnp.where(selb, sfx_b, 0))
                        - jnp.sum(jnp.where(selb, tv, 0)))
        if weighted:
            pltpu.sync_copy(sum_sh, wtotbuf)
            wv_b = plsc.load_gather(wtotbuf, [lanes, zeros_i])
            wsfx_b = jnp.flip(jnp.cumsum(jnp.flip(wv_b)))
            wabove_blocks = (jnp.sum(jnp.where(selb, wsfx_b, 0.0))
                             - jnp.sum(jnp.where(selb, wv_b, 0.0)))
        else:
            wabove_blocks = jnp.float32(0)

        @pl.when(sid == blk)
        @jax.named_scope(f"sc_scan{level}")
        def _owner():
            def ext_i(vec, idx):
                return jnp.sum(jnp.where(lanes == idx, vec, 0))

            def ext_f(vec, idx):
                return jnp.sum(jnp.where(lanes == idx, vec, 0.0))

            def cross(vec, need):
                sfx = jnp.flip(jnp.cumsum(jnp.flip(vec)))
                idx = jnp.max(plsc.all_reduce_population_count(
                    sfx >= need)) - 1
                above = ext_i(sfx, idx) - ext_i(vec, idx)
                return idx, above

            def wsuffix_above(vec, idx):
                wsfx = jnp.flip(jnp.cumsum(jnp.flip(vec)))
                return ext_f(wsfx, idx) - ext_f(vec, idx)

            need1 = needed - above_blocks
            def grp(g, gv):
                return jnp.where(lanes == g,
                                 jnp.sum(ctot[pl.ds(g * L, L)]), gv)
            gv = lax.fori_loop(0, L, grp, zeros_i)
            gs, above_g = cross(gv, need1)
            cv = ctot[pl.ds(gs * L, L)]
            need2 = need1 - above_g
            cs, above_c = cross(cv, need2)
            jc = gs * L + cs
            c = scanbuf[jc >> 5, pl.ds((jc & 31) * L, L)]
            need3 = need2 - above_c
            ls, above_l = cross(c, need3)
            binstar = jc * L + ls
            aboveg = above_blocks + above_g + above_c + above_l
            if weighted:
                def wgrp(g, gv):
                    return jnp.where(lanes == g,
                                     jnp.sum(wctot[pl.ds(g * L, L)]), gv)
                wgv = lax.fori_loop(0, L, wgrp, zeros_f)
                gbin = (blk * 4096 + jc * L) + lanes
                u_bin = (d1u << jnp.uint32(16)) | gbin.astype(jnp.uint32)
                w = c.astype(jnp.float32) * val_of(u_bin)
                waboveg = (wabove_blocks + wsuffix_above(wgv, gs)
                           + wsuffix_above(wctot[pl.ds(gs * L, L)], cs)
                           + wsuffix_above(w, ls))
            else:
                waboveg = jnp.float32(0)
            digit = blk * 4096 + binstar
            nxt = needed - aboveg
            wbits = plsc.bitcast(zeros_f + waboveg, jnp.int32)
            dvec = (jnp.where(lanes == 0, digit, 0)
                    + jnp.where(lanes == 1, nxt, 0)
                    + jnp.where(lanes == 2, wbits, 0))
            vec16[...] = dvec
            pltpu.sync_copy(vec16, dec_sh.at[level])

        with jax.named_scope(f"sc_dec{level}"):
            plsc.subcore_barrier()
        pltpu.sync_copy(dec_sh.at[level], decbuf)
        dv = decbuf[...]
        wdv = plsc.bitcast(dv, jnp.float32)
        w_above = jnp.sum(jnp.where(lanes == 2, wdv, 0.0))
        return dv[0], dv[1], w_above

    with jax.named_scope("ph_pass1"):
        zero_hist()
        stream_pass(p1, 0, first_h)
        h2 = pltpu.async_copy(keys_hbm.at[pl.ds(base, CHUNK)], buf0, sem0)
    with jax.named_scope("ph_sel1"):
        d1, needed2, _ = select_level(0, jnp.int32(K_TOP), jnp.uint32(0),
                                      hist_sh0)
    d1u = d1.astype(jnp.uint32)

    with jax.named_scope("ph_pass2"):
        zero_hist()
        accs = stream_pass(make_p2(d1u), (zeros_f,) * UNROLL, h2)
        acc_out = accs[0] + accs[1] + accs[2] + accs[3]
    with jax.named_scope("ph_sel2"):
        d2, needed_f, w_above = select_level(1, needed2, d1u, hist_sh1)

    t_u = (d1u << jnp.uint32(16)) | d2.astype(jnp.uint32)

    local = jnp.sum(acc_out)
    vec16f[...] = zeros_f + local
    pltpu.sync_copy(vec16f, sum_sh.at[sid])
    plsc.subcore_barrier()

    @pl.when(jnp.logical_and(sid == 0, cid == 0))
    def _emit():
        pltpu.sync_copy(sum_sh, wtotbuf)
        sums = plsc.load_gather(wtotbuf, [lanes, zeros_i])
        s_above = jnp.sum(sums) + w_above
        t_val = val_of(jnp.zeros((L,), jnp.uint32) + t_u)
        need_f = needed_f.astype(jnp.float32)
        res = (zeros_f + s_above + need_f * t_val) / jnp.float32(K_TOP)
        outbuf[...] = res
        pltpu.sync_copy(outbuf, out_hbm)


def _topk_mean(keys_flat):
    mesh = plsc.VectorSubcoreMesh(
        core_axis_name="c", subcore_axis_name="s", num_cores=2,
        num_subcores=NSUB)
    fn = functools.partial(
        pl.kernel,
        out_type=jax.ShapeDtypeStruct((L,), jnp.float32),
        mesh=mesh,
        compiler_params=pltpu.CompilerParams(
            use_tc_tiling_on_sc=False, needs_layout_passes=False),
        scratch_types=[
            pltpu.VMEM((CHUNK,), jnp.int32),
            pltpu.VMEM((CHUNK,), jnp.int32),
            pltpu.VMEM((HR, HC), jnp.int32),
            pltpu.VMEM((8, HC), jnp.int32),
            pltpu.VMEM((HR,), jnp.int32),
            pltpu.VMEM((L, L), jnp.int32),
            pltpu.VMEM((L, L), jnp.float32),
            pltpu.VMEM((L,), jnp.int32),
            pltpu.VMEM((L,), jnp.int32),
            pltpu.VMEM((L,), jnp.float32),
            pltpu.VMEM((L,), jnp.float32),
            pltpu.VMEM((8 * HC // L,), jnp.int32),
            pltpu.VMEM((8 * HC // L,), jnp.float32),
            pltpu.SemaphoreType.DMA,
            pltpu.SemaphoreType.DMA,
            pltpu.VMEM_SHARED((HR, HC), jnp.int32),
            pltpu.VMEM_SHARED((HR, HC), jnp.int32),
            pltpu.VMEM_SHARED((2, L, L), jnp.int32),
            pltpu.VMEM_SHARED((2, L), jnp.int32),
            pltpu.VMEM_SHARED((L, L), jnp.float32),
        ],
    )(_sc_body)
    return fn(keys_flat)


def kernel(inputs, targets):
    keys = _ce_keys(inputs, targets)
    out = _topk_mean(keys)
    return out[0]
